# Initial kernel scaffold; baseline (speedup 1.0000x reference)
import jax
import jax.numpy as jnp
from jax.experimental import pallas as pl


def kernel(x0, x1, x2, x3):
    raise NotImplementedError("write your pallas kernel here")


# speedup vs baseline: 2.6747x; 2.6747x over previous
import jax
import jax.numpy as jnp
from jax.experimental import pallas as pl
from jax.experimental.pallas import tpu as pltpu

B, H, N = 4, 12, 577
NT = N - 1
K = int(N * 0.5)


def _fused_kernel(v3_ref, x2_ref, x1_ref, x0_ref, mask_ref, vals_ref):
    h = pl.program_id(1)

    v = v3_ref[0, 0]
    for m_ref in (x2_ref, x1_ref, x0_ref):
        v_col = jnp.transpose(v)
        v = jnp.sum(v_col * m_ref[0, 0], axis=0, keepdims=True)
    tt = v[:, 1:]

    tt_col = jnp.transpose(tt)
    gt = tt_col > tt
    eq = tt_col == tt
    ii = jax.lax.broadcasted_iota(jnp.int32, (NT, NT), 0)
    jj = jax.lax.broadcasted_iota(jnp.int32, (NT, NT), 1)
    beats = gt | (eq & (ii < jj))
    rank = jnp.sum(beats.astype(jnp.float32), axis=0, keepdims=True)
    sel = (rank < float(K)).astype(jnp.float32)

    @pl.when(h == 0)
    def _():
        mask_ref[0] = sel

    @pl.when(h > 0)
    def _():
        mask_ref[0] = jnp.maximum(mask_ref[0], sel)

    @pl.when(h == H - 1)
    def _():
        rank_col = jnp.transpose(rank)
        rr = jax.lax.broadcasted_iota(jnp.int32, (NT, K), 1).astype(jnp.float32)
        onehot = (rank_col == rr).astype(jnp.float32)
        vals_ref[0] = jnp.dot(tt, onehot, preferred_element_type=jnp.float32)


def kernel(x0, x1, x2, x3):
    v3 = x3[:, :, 0:1, :]

    mask_f32, vals = pl.pallas_call(
        _fused_kernel,
        grid=(B, H),
        in_specs=[
            pl.BlockSpec((1, 1, 1, N), lambda b, h: (b, h, 0, 0)),
            pl.BlockSpec((1, 1, N, N), lambda b, h: (b, h, 0, 0)),
            pl.BlockSpec((1, 1, N, N), lambda b, h: (b, h, 0, 0)),
            pl.BlockSpec((1, 1, N, N), lambda b, h: (b, h, 0, 0)),
        ],
        out_specs=[
            pl.BlockSpec((1, 1, NT), lambda b, h: (b, 0, 0)),
            pl.BlockSpec((1, 1, K), lambda b, h: (b, 0, 0)),
        ],
        out_shape=[
            jax.ShapeDtypeStruct((B, 1, NT), jnp.float32),
            jax.ShapeDtypeStruct((B, 1, K), jnp.float32),
        ],
    )(v3, x2, x1, x0)

    mask = jnp.concatenate(
        [mask_f32[:, 0, :] != 0.0, jnp.zeros((B, 1), dtype=bool)], axis=1)
    return vals[:, 0, :], mask
